# baseline (speedup 1.0000x reference)
import functools

import jax
import jax.numpy as jnp
from jax.experimental import pallas as pl
from jax.experimental.pallas import tpu as pltpu

N = 4096
IN_F = 128
OUT_F = 128
H = 16
ATOM = OUT_F // H
BM = 1024


def _body(x_ref, w_ref, a_ref, e_ref, adj_ref, out_ref, bw_ref, v_ref):
    @pl.when(pl.program_id(0) == 0)
    def _prologue():
        mx = jnp.dot(x_ref[...], w_ref[...], preferred_element_type=jnp.float32)
        s = jnp.dot(mx, a_ref[...], preferred_element_type=jnp.float32)
        v = jnp.exp(s)
        v_ref[...] = v
        v_exp = jnp.dot(v, e_ref[...], preferred_element_type=jnp.float32)
        bw_ref[...] = mx * v_exp

    adj_blk = adj_ref[...]
    num = jax.lax.dot_general(adj_blk, bw_ref[...], (((1,), (0,)), ((), ())),
                              precision=jax.lax.Precision.DEFAULT,
                              preferred_element_type=jnp.float32)
    den = jax.lax.dot_general(adj_blk, v_ref[...], (((1,), (0,)), ((), ())),
                              precision=jax.lax.Precision.DEFAULT,
                              preferred_element_type=jnp.float32)
    den_exp = jnp.dot(den, e_ref[...], preferred_element_type=jnp.float32)
    out_ref[...] = num / den_exp


@functools.partial(jax.jit, static_argnames=("interpret",))
def kernel(x, adj, W, alpha_origin, alpha, interpret=False):
    del alpha_origin
    w_cat = W.transpose(1, 0, 2).reshape(IN_F, H * ATOM)
    a_mat = (jnp.eye(H, dtype=jnp.float32)[:, None, :] * alpha[:, :, None]).reshape(
        H * ATOM, H
    )
    e_mat = jnp.repeat(jnp.eye(H, dtype=jnp.float32), ATOM, axis=1)

    grid = (N // BM,)
    return pl.pallas_call(
        _body,
        grid=grid,
        in_specs=[
            pl.BlockSpec((N, IN_F), lambda i: (0, 0)),
            pl.BlockSpec((IN_F, H * ATOM), lambda i: (0, 0)),
            pl.BlockSpec((H * ATOM, H), lambda i: (0, 0)),
            pl.BlockSpec((H, H * ATOM), lambda i: (0, 0)),
            pl.BlockSpec((BM, N), lambda i: (i, 0)),
        ],
        out_specs=pl.BlockSpec((BM, OUT_F), lambda i: (i, 0)),
        out_shape=jax.ShapeDtypeStruct((N, OUT_F), jnp.float32),
        scratch_shapes=[
            pltpu.VMEM((N, H * ATOM), jnp.float32),
            pltpu.VMEM((N, H), jnp.float32),
        ],
        interpret=interpret,
    )(x, w_cat, a_mat, e_mat, adj)


# speedup vs baseline: 1.1217x; 1.1217x over previous
import functools

import jax
import jax.numpy as jnp
from jax.experimental import pallas as pl
from jax.experimental.pallas import tpu as pltpu

N = 4096
IN_F = 128
OUT_F = 128
H = 16
ATOM = OUT_F // H
BM = 1024


def _body(x_ref, w_ref, a_ref, e_ref, adj_ref, out_ref, b_ref):
    @pl.when(pl.program_id(0) == 0)
    def _prologue():
        mx = jnp.dot(x_ref[...], w_ref[...], preferred_element_type=jnp.float32)
        s = jnp.dot(mx, a_ref[...], preferred_element_type=jnp.float32)
        v = jnp.exp(s)
        v_exp = jnp.dot(v, e_ref[...], preferred_element_type=jnp.float32)
        b_ref[:, : H * ATOM] = mx * v_exp
        b_ref[:, H * ATOM :] = v

    res = jax.lax.dot_general(adj_ref[...], b_ref[...], (((1,), (0,)), ((), ())),
                              precision=jax.lax.Precision.DEFAULT,
                              preferred_element_type=jnp.float32)
    den_exp = jnp.dot(res[:, H * ATOM :], e_ref[...],
                      preferred_element_type=jnp.float32)
    out_ref[...] = res[:, : H * ATOM] / den_exp


@functools.partial(jax.jit, static_argnames=("interpret",))
def kernel(x, adj, W, alpha_origin, alpha, interpret=False):
    del alpha_origin
    w_cat = W.transpose(1, 0, 2).reshape(IN_F, H * ATOM)
    a_mat = (jnp.eye(H, dtype=jnp.float32)[:, None, :] * alpha[:, :, None]).reshape(
        H * ATOM, H
    )
    e_mat = jnp.repeat(jnp.eye(H, dtype=jnp.float32), ATOM, axis=1)

    grid = (N // BM,)
    return pl.pallas_call(
        _body,
        grid=grid,
        in_specs=[
            pl.BlockSpec((N, IN_F), lambda i: (0, 0)),
            pl.BlockSpec((IN_F, H * ATOM), lambda i: (0, 0)),
            pl.BlockSpec((H * ATOM, H), lambda i: (0, 0)),
            pl.BlockSpec((H, H * ATOM), lambda i: (0, 0)),
            pl.BlockSpec((BM, N), lambda i: (i, 0)),
        ],
        out_specs=pl.BlockSpec((BM, OUT_F), lambda i: (i, 0)),
        out_shape=jax.ShapeDtypeStruct((N, OUT_F), jnp.float32),
        scratch_shapes=[
            pltpu.VMEM((N, H * ATOM + H), jnp.float32),
        ],
        interpret=interpret,
    )(x, w_cat, a_mat, e_mat, adj)


# speedup vs baseline: 1.1273x; 1.0049x over previous
import functools

import jax
import jax.numpy as jnp
from jax.experimental import pallas as pl
from jax.experimental.pallas import tpu as pltpu

N = 4096
IN_F = 128
OUT_F = 128
H = 16
ATOM = OUT_F // H
BM = 512


def _body(x_ref, w_ref, a_ref, e_ref, adj_ref, out_ref, b_ref):
    @pl.when(pl.program_id(0) == 0)
    def _prologue():
        mx = jnp.dot(x_ref[...], w_ref[...], preferred_element_type=jnp.float32)
        s = jnp.dot(mx, a_ref[...], preferred_element_type=jnp.float32)
        v = jnp.exp(s)
        v_exp = jnp.dot(v, e_ref[...], preferred_element_type=jnp.float32)
        b_ref[:, : H * ATOM] = mx * v_exp
        b_ref[:, H * ATOM :] = v

    res = jax.lax.dot_general(adj_ref[...], b_ref[...], (((1,), (0,)), ((), ())),
                              precision=jax.lax.Precision.DEFAULT,
                              preferred_element_type=jnp.float32)
    den_exp = jnp.dot(res[:, H * ATOM :], e_ref[...],
                      preferred_element_type=jnp.float32)
    out_ref[...] = res[:, : H * ATOM] / den_exp


@functools.partial(jax.jit, static_argnames=("interpret",))
def kernel(x, adj, W, alpha_origin, alpha, interpret=False):
    del alpha_origin
    w_cat = W.transpose(1, 0, 2).reshape(IN_F, H * ATOM)
    a_mat = (jnp.eye(H, dtype=jnp.float32)[:, None, :] * alpha[:, :, None]).reshape(
        H * ATOM, H
    )
    e_mat = jnp.repeat(jnp.eye(H, dtype=jnp.float32), ATOM, axis=1)

    grid = (N // BM,)
    return pl.pallas_call(
        _body,
        grid=grid,
        in_specs=[
            pl.BlockSpec((N, IN_F), lambda i: (0, 0)),
            pl.BlockSpec((IN_F, H * ATOM), lambda i: (0, 0)),
            pl.BlockSpec((H * ATOM, H), lambda i: (0, 0)),
            pl.BlockSpec((H, H * ATOM), lambda i: (0, 0)),
            pl.BlockSpec((BM, N), lambda i: (i, 0)),
        ],
        out_specs=pl.BlockSpec((BM, OUT_F), lambda i: (i, 0)),
        out_shape=jax.ShapeDtypeStruct((N, OUT_F), jnp.float32),
        scratch_shapes=[
            pltpu.VMEM((N, H * ATOM + H), jnp.float32),
        ],
        interpret=interpret,
    )(x, w_cat, a_mat, e_mat, adj)
